# Initial kernel scaffold; baseline (speedup 1.0000x reference)
import jax
import jax.numpy as jnp
from jax.experimental import pallas as pl


def kernel(x, edge_index, params):
    raise NotImplementedError("write your pallas kernel here")


# speedup vs baseline: 29.7949x; 29.7949x over previous
import jax
import jax.numpy as jnp
from functools import partial
from jax.experimental import pallas as pl

_F32 = jnp.float32
_HI = jax.lax.Precision.HIGHEST


def _dot(a, b, dims=None, precision=_HI):
    if dims is None:
        dims = (((1,), (0,)), ((), ()))
    return jax.lax.dot_general(a, b, dimension_numbers=dims,
                               precision=precision,
                               preferred_element_type=_F32)


def _softmax0(x):
    m = jnp.max(x, axis=0, keepdims=True)
    e = jnp.exp(x - m)
    return e / jnp.sum(e, axis=0, keepdims=True)


def _mbuild_kernel(ei_ref, m_ref, *, n, n_chunks, chunk):
    iota = jax.lax.broadcasted_iota(jnp.int32, (n, chunk), 0)
    deg = jnp.zeros((n, 1), _F32)
    for k in range(n_chunks):
        rows = ei_ref[0, k:k + 1, :]
        cols = ei_ref[1, k:k + 1, :]
        mask = (rows != cols).astype(_F32)
        rm = (iota == rows).astype(_F32)
        deg = deg + jnp.sum(rm * mask, axis=1, keepdims=True)
    dis = jnp.where(deg > 0, jax.lax.rsqrt(jnp.maximum(deg, 1e-12)), 0.0)
    macc = jnp.zeros((n, n), _F32)
    for k in range(n_chunks):
        rows = ei_ref[0, k:k + 1, :]
        cols = ei_ref[1, k:k + 1, :]
        mask = (rows != cols).astype(_F32)
        rm = (iota == rows).astype(_F32)
        cm = (iota == cols).astype(_F32)
        dis_r = jnp.sum(rm * dis, axis=0, keepdims=True)
        dis_c = jnp.sum(cm * dis, axis=0, keepdims=True)
        norm = -mask * dis_r * dis_c
        macc = macc + _dot(rm * norm, cm, dims=(((1,), (1,)), ((), ())))
    m_ref[:, :] = macc


def _build_m(edge_index, n):
    e = edge_index.shape[1]
    chunk = 128
    n_chunks = e // chunk
    ei3 = edge_index.reshape(2, n_chunks, chunk)
    return pl.pallas_call(
        partial(_mbuild_kernel, n=n, n_chunks=n_chunks, chunk=chunk),
        out_shape=jax.ShapeDtypeStruct((n, n), _F32),
    )(ei3)


def _block_kernel(x_ref, m_ref,
                  u1_ref, u2_ref, u3_ref, be_ref, ve_ref,
                  w1_ref, w2_ref, w3_ref, bs_ref, vs_ref,
                  cw0_ref, cw1_ref, cw2_ref, cb_ref,
                  tw0_ref, tw1_ref, tw2_ref, tb_ref,
                  rw_ref, rb_ref, lg_ref, lb_ref,
                  out_ref, *, t, n, f, c):
    X = x_ref[0]
    M = m_ref[:, :]

    u1 = u1_ref[:, :]
    lhs_tf = jnp.concatenate(
        [_dot(u1, X[ti * n:(ti + 1) * n, :]) for ti in range(t)], axis=0)
    lhs = _dot(lhs_tf, u2_ref[:, :])
    rhs_flat = _dot(X, u3_ref[:, :])
    E = jnp.concatenate(
        [_dot(lhs, rhs_flat[si * n:(si + 1) * n, :]) for si in range(t)],
        axis=1)
    Eatt = _dot(ve_ref[:, :], jax.nn.sigmoid(E + be_ref[:, :]))
    Eatt = _softmax0(Eatt)

    Xt = jnp.concatenate([
        sum(Eatt[ti:ti + 1, si:si + 1] * X[ti * n:(ti + 1) * n, :]
            for ti in range(t))
        for si in range(t)], axis=0)

    lhs2 = sum(w1_ref[si:si + 1, 0:1] * Xt[si * n:(si + 1) * n, :]
               for si in range(t))
    lhs2b = _dot(lhs2, w2_ref[:, :])
    rhs2_flat = _dot(Xt, w3_ref[:, :])
    rhs2 = jnp.concatenate(
        [rhs2_flat[ti * n:(ti + 1) * n, :] for ti in range(t)], axis=1)
    P = _dot(lhs2b, rhs2, dims=(((1,), (1,)), ((), ())))
    S = _dot(vs_ref[:, :], jax.nn.sigmoid(P + bs_ref[:, :]))
    S = _softmax0(S)

    eye = (jax.lax.broadcasted_iota(jnp.int32, (n, n), 0) ==
           jax.lax.broadcasted_iota(jnp.int32, (n, n), 1)).astype(_F32)
    diag = jnp.sum(S * eye, axis=1, keepdims=True)
    diag_full = jnp.concatenate([diag] * t, axis=0)
    A1 = M * S

    T0 = diag_full * Xt
    o = _dot(T0, cw0_ref[:, :])
    T1 = jnp.concatenate(
        [_dot(A1, T0[ti * n:(ti + 1) * n, :]) for ti in range(t)], axis=0)
    o = o + _dot(T1, cw1_ref[:, :])
    T2 = jnp.concatenate(
        [_dot(M, T1[ti * n:(ti + 1) * n, :]) for ti in range(t)], axis=0)
    T2 = 2.0 * T2 - T0
    o = o + _dot(T2, cw2_ref[:, :]) + cb_ref[:, :]
    Xh = jax.nn.relu(o)

    Z0 = _dot(Xh, tw0_ref[:, :])
    Z1 = _dot(Xh, tw1_ref[:, :])
    Z2 = _dot(Xh, tw2_ref[:, :])
    zpad = jnp.zeros((n, c), _F32)
    tc = (Z1 + jnp.concatenate([zpad, Z0[:(t - 1) * n, :]], axis=0)
          + jnp.concatenate([Z2[n:, :], zpad], axis=0) + tb_ref[:, :])
    rc = _dot(X, rw_ref[:, :]) + rb_ref[:, :]
    z = jax.nn.relu(rc + tc)
    mu = jnp.mean(z, axis=1, keepdims=True)
    zc = z - mu
    var = jnp.mean(zc * zc, axis=1, keepdims=True)
    z = zc * jax.lax.rsqrt(var + 1e-5) * lg_ref[:, :] + lb_ref[:, :]
    out_ref[0] = z


def _run_block(X, M, p, t, n, c):
    b = X.shape[0]
    f = X.shape[2]
    prm = [
        p['U1'].reshape(1, n), p['U2'], p['U3'].reshape(f, 1),
        p['be'][0], p['Ve'],
        p['W1'].reshape(t, 1), p['W2'], p['W3'].reshape(f, 1),
        p['bs'][0], p['Vs'],
        p['cheb_w'][0], p['cheb_w'][1], p['cheb_w'][2],
        p['cheb_b'].reshape(1, c),
        p['time_w'][:, :, 0, 0].T, p['time_w'][:, :, 0, 1].T,
        p['time_w'][:, :, 0, 2].T, p['time_b'].reshape(1, c),
        p['res_w'][:, :, 0, 0].T, p['res_b'].reshape(1, c),
        p['ln_g'].reshape(1, c), p['ln_b'].reshape(1, c),
    ]
    full = lambda a: pl.BlockSpec(a.shape, lambda i: (0,) * a.ndim)
    in_specs = ([pl.BlockSpec((1, t * n, f), lambda i: (i, 0, 0)),
                 pl.BlockSpec((n, n), lambda i: (0, 0))] +
                [full(a) for a in prm])
    return pl.pallas_call(
        partial(_block_kernel, t=t, n=n, f=f, c=c),
        grid=(b,),
        in_specs=in_specs,
        out_specs=pl.BlockSpec((1, t * n, c), lambda i: (i, 0, 0)),
        out_shape=jax.ShapeDtypeStruct((b, t * n, c), _F32),
    )(X, M, *prm)


def _final_kernel(x_ref, fw_ref, fcw_ref, fcb_ref, out_ref, *, t, n, c):
    X = x_ref[0]
    fw = fw_ref[:, :]
    o = sum(_dot(X[ti * n:(ti + 1) * n, :], fw[:, ti:ti + 1])
            for ti in range(t))
    y = _dot(fcw_ref[:, :], o) + fcb_ref[:, :]
    out_ref[0] = jax.nn.sigmoid(y)


def _run_final(X, params, t, n, c):
    b = X.shape[0]
    fw = params['final_w'][0, :, 0, :].T
    fcb = (params['fc_b'] +
           params['final_b'][0] * params['fc_w'].sum(axis=1)).reshape(n, 1)
    full = lambda a: pl.BlockSpec(a.shape, lambda i: (0,) * a.ndim)
    out = pl.pallas_call(
        partial(_final_kernel, t=t, n=n, c=c),
        grid=(b,),
        in_specs=[pl.BlockSpec((1, t * n, c), lambda i: (i, 0, 0)),
                  full(fw), full(params['fc_w']), full(fcb)],
        out_specs=pl.BlockSpec((1, n, 1), lambda i: (i, 0, 0)),
        out_shape=jax.ShapeDtypeStruct((b, n, 1), _F32),
    )(X, fw, params['fc_w'], fcb)
    return jnp.transpose(out, (0, 2, 1))


def kernel(x, edge_index, params):
    b, t, n, f = x.shape
    c = params['blocks'][0]['cheb_b'].shape[0]
    M = _build_m(edge_index, n)
    X = x.reshape(b, t * n, f)
    for p in params['blocks']:
        X = _run_block(X, M, p, t, n, c)
    return _run_final(X, params, t, n, c)


# speedup vs baseline: 30.8020x; 1.0338x over previous
import jax
import jax.numpy as jnp
from functools import partial
from jax.experimental import pallas as pl
from jax.experimental.pallas import tpu as pltpu
from jax.experimental.pallas import tpu_sc as plsc

_F32 = jnp.float32
_HI = jax.lax.Precision.HIGHEST


def _dot(a, b, dims=None, precision=_HI):
    if dims is None:
        dims = (((1,), (0,)), ((), ()))
    return jax.lax.dot_general(a, b, dimension_numbers=dims,
                               precision=precision,
                               preferred_element_type=_F32)


def _softmax0(x):
    m = jnp.max(x, axis=0, keepdims=True)
    e = jnp.exp(x - m)
    return e / jnp.sum(e, axis=0, keepdims=True)


_NC, _NS, _LANES = 2, 16, 16
_ROWW = 128


def _cnt_sc_kernel(rows_hbm, cols_hbm, onehot_hbm, out_hbm,
                   rows_v, cols_v, buf_v, ridx_v, gidx_v, zbuf_v, shared,
                   *, n, e_per_w):
    cid = jax.lax.axis_index("c")
    sid = jax.lax.axis_index("s")
    wid = cid * _NS + sid
    nrows = (n * n) // _ROWW
    sl = nrows // _NS
    zn = zbuf_v.shape[0]
    groups = e_per_w // _LANES

    z16 = jnp.zeros((_LANES,), _F32)
    for r in range(zn):
        for q in range(_ROWW // _LANES):
            zbuf_v[r, pl.ds(q * _LANES, _LANES)] = z16
    for k in range(sl // zn):
        pltpu.sync_copy(zbuf_v, shared.at[pl.ds(sid * sl + k * zn, zn)])

    pltpu.sync_copy(rows_hbm.at[wid], rows_v)
    pltpu.sync_copy(cols_hbm.at[wid], cols_v)

    for g in range(groups):
        rr = rows_v[pl.ds(g * _LANES, _LANES)]
        cc = cols_v[pl.ds(g * _LANES, _LANES)]
        flat = rr * n + cc
        rident = jax.lax.shift_right_logical(flat, 7)
        lane = jax.lax.bitwise_and(flat, _ROWW - 1)
        sel = jnp.where(rr != cc, lane, _ROWW)
        ridx_v[g // 8, pl.ds((g % 8) * _LANES, _LANES)] = rident
        gidx_v[g // 8, pl.ds((g % 8) * _LANES, _LANES)] = sel

    for j in range(e_per_w // 128):
        pltpu.sync_copy(onehot_hbm.at[gidx_v.at[j]],
                        buf_v.at[pl.ds(j * 128, 128)])
    plsc.subcore_barrier()
    for j in range(e_per_w // 128):
        pltpu.sync_copy(buf_v.at[pl.ds(j * 128, 128)],
                        shared.at[ridx_v.at[j]], add=True)
    plsc.subcore_barrier()
    pltpu.sync_copy(shared.at[pl.ds(sid * sl, sl)],
                    out_hbm.at[cid, pl.ds(sid * sl, sl)])


def _build_cnt(edge_index, n):
    e = edge_index.shape[1]
    nw = _NC * _NS
    e_per_w = e // nw
    nrows = (n * n) // _ROWW
    rows32 = edge_index[0].reshape(nw, e_per_w)
    cols32 = edge_index[1].reshape(nw, e_per_w)
    onehot = jnp.concatenate(
        [jnp.eye(_ROWW, dtype=_F32), jnp.zeros((8, _ROWW), _F32)], axis=0)
    mesh = plsc.VectorSubcoreMesh(core_axis_name="c", subcore_axis_name="s")
    k = partial(
        pl.kernel,
        mesh=mesh,
        out_type=jax.ShapeDtypeStruct((_NC, nrows, _ROWW), _F32),
        scratch_types=[
            pltpu.VMEM((e_per_w,), jnp.int32),
            pltpu.VMEM((e_per_w,), jnp.int32),
            pltpu.VMEM((e_per_w, _ROWW), _F32),
            pltpu.VMEM((e_per_w // 128, 128), jnp.int32),
            pltpu.VMEM((e_per_w // 128, 128), jnp.int32),
            pltpu.VMEM((64, _ROWW), _F32),
            pltpu.VMEM_SHARED((nrows, _ROWW), _F32),
        ],
    )(partial(_cnt_sc_kernel, n=n, e_per_w=e_per_w))
    cnt2 = k(rows32, cols32, onehot)
    return cnt2.reshape(_NC, n, n)


def _mfin_kernel(cnt_ref, m_ref, *, n):
    cnt = cnt_ref[0] + cnt_ref[1]
    deg = jnp.sum(cnt, axis=1, keepdims=True)
    dis = jnp.where(deg > 0, jax.lax.rsqrt(jnp.maximum(deg, 1e-12)), 0.0)
    eye = (jax.lax.broadcasted_iota(jnp.int32, (n, n), 0) ==
           jax.lax.broadcasted_iota(jnp.int32, (n, n), 1)).astype(_F32)
    dis_row = _dot(dis, eye, dims=(((0,), (0,)), ((), ())))
    m_ref[:, :] = -(cnt * dis) * dis_row


def _build_m(edge_index, n):
    cnt2 = _build_cnt(edge_index, n)
    return pl.pallas_call(
        partial(_mfin_kernel, n=n),
        out_shape=jax.ShapeDtypeStruct((n, n), _F32),
    )(cnt2)


def _block_kernel(x_ref, m_ref,
                  u1_ref, u2_ref, u3_ref, be_ref, ve_ref,
                  w1_ref, w2_ref, w3_ref, bs_ref, vs_ref,
                  cw0_ref, cw1_ref, cw2_ref, cb_ref,
                  tw0_ref, tw1_ref, tw2_ref, tb_ref,
                  rw_ref, rb_ref, lg_ref, lb_ref,
                  out_ref, *, t, n, f, c):
    X = x_ref[0]
    M = m_ref[:, :]

    u1 = u1_ref[:, :]
    lhs_tf = jnp.concatenate(
        [_dot(u1, X[ti * n:(ti + 1) * n, :]) for ti in range(t)], axis=0)
    lhs = _dot(lhs_tf, u2_ref[:, :])
    rhs_flat = _dot(X, u3_ref[:, :])
    E = jnp.concatenate(
        [_dot(lhs, rhs_flat[si * n:(si + 1) * n, :]) for si in range(t)],
        axis=1)
    Eatt = _dot(ve_ref[:, :], jax.nn.sigmoid(E + be_ref[:, :]))
    Eatt = _softmax0(Eatt)

    Xt = jnp.concatenate([
        sum(Eatt[ti:ti + 1, si:si + 1] * X[ti * n:(ti + 1) * n, :]
            for ti in range(t))
        for si in range(t)], axis=0)

    lhs2 = sum(w1_ref[si:si + 1, 0:1] * Xt[si * n:(si + 1) * n, :]
               for si in range(t))
    lhs2b = _dot(lhs2, w2_ref[:, :])
    rhs2_flat = _dot(Xt, w3_ref[:, :])
    rhs2 = jnp.concatenate(
        [rhs2_flat[ti * n:(ti + 1) * n, :] for ti in range(t)], axis=1)
    P = _dot(lhs2b, rhs2, dims=(((1,), (1,)), ((), ())))
    S = _dot(vs_ref[:, :], jax.nn.sigmoid(P + bs_ref[:, :]))
    S = _softmax0(S)

    eye = (jax.lax.broadcasted_iota(jnp.int32, (n, n), 0) ==
           jax.lax.broadcasted_iota(jnp.int32, (n, n), 1)).astype(_F32)
    diag = jnp.sum(S * eye, axis=1, keepdims=True)
    diag_full = jnp.concatenate([diag] * t, axis=0)
    A1 = M * S

    T0 = diag_full * Xt
    o = _dot(T0, cw0_ref[:, :])
    T1 = jnp.concatenate(
        [_dot(A1, T0[ti * n:(ti + 1) * n, :]) for ti in range(t)], axis=0)
    o = o + _dot(T1, cw1_ref[:, :])
    T2 = jnp.concatenate(
        [_dot(M, T1[ti * n:(ti + 1) * n, :]) for ti in range(t)], axis=0)
    T2 = 2.0 * T2 - T0
    o = o + _dot(T2, cw2_ref[:, :]) + cb_ref[:, :]
    Xh = jax.nn.relu(o)

    Z0 = _dot(Xh, tw0_ref[:, :])
    Z1 = _dot(Xh, tw1_ref[:, :])
    Z2 = _dot(Xh, tw2_ref[:, :])
    zpad = jnp.zeros((n, c), _F32)
    tc = (Z1 + jnp.concatenate([zpad, Z0[:(t - 1) * n, :]], axis=0)
          + jnp.concatenate([Z2[n:, :], zpad], axis=0) + tb_ref[:, :])
    rc = _dot(X, rw_ref[:, :]) + rb_ref[:, :]
    z = jax.nn.relu(rc + tc)
    mu = jnp.mean(z, axis=1, keepdims=True)
    zc = z - mu
    var = jnp.mean(zc * zc, axis=1, keepdims=True)
    z = zc * jax.lax.rsqrt(var + 1e-5) * lg_ref[:, :] + lb_ref[:, :]
    out_ref[0] = z


def _run_block(X, M, p, t, n, c):
    b = X.shape[0]
    f = X.shape[2]
    prm = [
        p['U1'].reshape(1, n), p['U2'], p['U3'].reshape(f, 1),
        p['be'][0], p['Ve'],
        p['W1'].reshape(t, 1), p['W2'], p['W3'].reshape(f, 1),
        p['bs'][0], p['Vs'],
        p['cheb_w'][0], p['cheb_w'][1], p['cheb_w'][2],
        p['cheb_b'].reshape(1, c),
        p['time_w'][:, :, 0, 0].T, p['time_w'][:, :, 0, 1].T,
        p['time_w'][:, :, 0, 2].T, p['time_b'].reshape(1, c),
        p['res_w'][:, :, 0, 0].T, p['res_b'].reshape(1, c),
        p['ln_g'].reshape(1, c), p['ln_b'].reshape(1, c),
    ]
    full = lambda a: pl.BlockSpec(a.shape, lambda i: (0,) * a.ndim)
    in_specs = ([pl.BlockSpec((1, t * n, f), lambda i: (i, 0, 0)),
                 pl.BlockSpec((n, n), lambda i: (0, 0))] +
                [full(a) for a in prm])
    return pl.pallas_call(
        partial(_block_kernel, t=t, n=n, f=f, c=c),
        grid=(b,),
        in_specs=in_specs,
        out_specs=pl.BlockSpec((1, t * n, c), lambda i: (i, 0, 0)),
        out_shape=jax.ShapeDtypeStruct((b, t * n, c), _F32),
    )(X, M, *prm)


def _final_kernel(x_ref, fw_ref, fcw_ref, fcb_ref, out_ref, *, t, n, c):
    X = x_ref[0]
    fw = fw_ref[:, :]
    o = sum(_dot(X[ti * n:(ti + 1) * n, :], fw[:, ti:ti + 1])
            for ti in range(t))
    y = _dot(fcw_ref[:, :], o) + fcb_ref[:, :]
    out_ref[0] = jax.nn.sigmoid(y)


def _run_final(X, params, t, n, c):
    b = X.shape[0]
    fw = params['final_w'][0, :, 0, :].T
    fcb = (params['fc_b'] +
           params['final_b'][0] * params['fc_w'].sum(axis=1)).reshape(n, 1)
    full = lambda a: pl.BlockSpec(a.shape, lambda i: (0,) * a.ndim)
    out = pl.pallas_call(
        partial(_final_kernel, t=t, n=n, c=c),
        grid=(b,),
        in_specs=[pl.BlockSpec((1, t * n, c), lambda i: (i, 0, 0)),
                  full(fw), full(params['fc_w']), full(fcb)],
        out_specs=pl.BlockSpec((1, n, 1), lambda i: (i, 0, 0)),
        out_shape=jax.ShapeDtypeStruct((b, n, 1), _F32),
    )(X, fw, params['fc_w'], fcb)
    return jnp.transpose(out, (0, 2, 1))


def kernel(x, edge_index, params):
    b, t, n, f = x.shape
    c = params['blocks'][0]['cheb_b'].shape[0]
    M = _build_m(edge_index, n)
    X = x.reshape(b, t * n, f)
    for p in params['blocks']:
        X = _run_block(X, M, p, t, n, c)
    return _run_final(X, params, t, n, c)


# speedup vs baseline: 30.8508x; 1.0016x over previous
import jax
import jax.numpy as jnp
from functools import partial
from jax.experimental import pallas as pl
from jax.experimental.pallas import tpu as pltpu
from jax.experimental.pallas import tpu_sc as plsc

_F32 = jnp.float32
_HI = jax.lax.Precision.HIGHEST


def _dot(a, b, dims=None, precision=_HI):
    if dims is None:
        dims = (((1,), (0,)), ((), ()))
    return jax.lax.dot_general(a, b, dimension_numbers=dims,
                               precision=precision,
                               preferred_element_type=_F32)


def _softmax0(x):
    m = jnp.max(x, axis=0, keepdims=True)
    e = jnp.exp(x - m)
    return e / jnp.sum(e, axis=0, keepdims=True)


_NC, _NS, _LANES = 2, 16, 16
_ROWW = 128


def _cnt_sc_kernel(rows_hbm, cols_hbm, onehot_hbm, out_hbm,
                   rows_v, cols_v, buf_v, ridx_v, gidx_v, zbuf_v, shared,
                   *, n, e_per_w):
    cid = jax.lax.axis_index("c")
    sid = jax.lax.axis_index("s")
    wid = cid * _NS + sid
    nrows = (n * n) // _ROWW
    sl = nrows // _NS
    zn = zbuf_v.shape[0]
    groups = e_per_w // _LANES

    z16 = jnp.zeros((_LANES,), _F32)
    for r in range(zn):
        for q in range(_ROWW // _LANES):
            zbuf_v[r, pl.ds(q * _LANES, _LANES)] = z16
    for k in range(sl // zn):
        pltpu.sync_copy(zbuf_v, shared.at[pl.ds(sid * sl + k * zn, zn)])

    pltpu.sync_copy(rows_hbm.at[wid], rows_v)
    pltpu.sync_copy(cols_hbm.at[wid], cols_v)

    for g in range(groups):
        rr = rows_v[pl.ds(g * _LANES, _LANES)]
        cc = cols_v[pl.ds(g * _LANES, _LANES)]
        flat = rr * n + cc
        rident = jax.lax.shift_right_logical(flat, 7)
        lane = jax.lax.bitwise_and(flat, _ROWW - 1)
        sel = jnp.where(rr != cc, lane, _ROWW)
        ridx_v[g // 8, pl.ds((g % 8) * _LANES, _LANES)] = rident
        gidx_v[g // 8, pl.ds((g % 8) * _LANES, _LANES)] = sel

    for j in range(e_per_w // 128):
        pltpu.sync_copy(onehot_hbm.at[gidx_v.at[j]],
                        buf_v.at[pl.ds(j * 128, 128)])
    plsc.subcore_barrier()
    for j in range(e_per_w // 128):
        pltpu.sync_copy(buf_v.at[pl.ds(j * 128, 128)],
                        shared.at[ridx_v.at[j]], add=True)
    plsc.subcore_barrier()
    pltpu.sync_copy(shared.at[pl.ds(sid * sl, sl)],
                    out_hbm.at[cid, pl.ds(sid * sl, sl)])


def _build_cnt(edge_index, n):
    e = edge_index.shape[1]
    nw = _NC * _NS
    e_per_w = e // nw
    nrows = (n * n) // _ROWW
    rows32 = edge_index[0].reshape(nw, e_per_w)
    cols32 = edge_index[1].reshape(nw, e_per_w)
    onehot = jnp.concatenate(
        [jnp.eye(_ROWW, dtype=_F32), jnp.zeros((8, _ROWW), _F32)], axis=0)
    mesh = plsc.VectorSubcoreMesh(core_axis_name="c", subcore_axis_name="s")
    k = partial(
        pl.kernel,
        mesh=mesh,
        out_type=jax.ShapeDtypeStruct((_NC, nrows, _ROWW), _F32),
        scratch_types=[
            pltpu.VMEM((e_per_w,), jnp.int32),
            pltpu.VMEM((e_per_w,), jnp.int32),
            pltpu.VMEM((e_per_w, _ROWW), _F32),
            pltpu.VMEM((e_per_w // 128, 128), jnp.int32),
            pltpu.VMEM((e_per_w // 128, 128), jnp.int32),
            pltpu.VMEM((64, _ROWW), _F32),
            pltpu.VMEM_SHARED((nrows, _ROWW), _F32),
        ],
    )(partial(_cnt_sc_kernel, n=n, e_per_w=e_per_w))
    cnt2 = k(rows32, cols32, onehot)
    return cnt2.reshape(_NC, n, n)


def _mfin_kernel(cnt_ref, m_ref, *, n):
    cnt = cnt_ref[0] + cnt_ref[1]
    deg = jnp.sum(cnt, axis=1, keepdims=True)
    dis = jnp.where(deg > 0, jax.lax.rsqrt(jnp.maximum(deg, 1e-12)), 0.0)
    eye = (jax.lax.broadcasted_iota(jnp.int32, (n, n), 0) ==
           jax.lax.broadcasted_iota(jnp.int32, (n, n), 1)).astype(_F32)
    dis_row = _dot(dis, eye, dims=(((0,), (0,)), ((), ())))
    m_ref[:, :] = -(cnt * dis) * dis_row


def _build_m(edge_index, n):
    cnt2 = _build_cnt(edge_index, n)
    return pl.pallas_call(
        partial(_mfin_kernel, n=n),
        out_shape=jax.ShapeDtypeStruct((n, n), _F32),
    )(cnt2)


def _block_kernel(x_ref, m_ref,
                  u1_ref, u2_ref, u3_ref, be_ref, ve_ref,
                  w1_ref, w2_ref, w3_ref, bs_ref, vs_ref,
                  cw0_ref, cw1_ref, cw2_ref, cb_ref,
                  tw0_ref, tw1_ref, tw2_ref, tb_ref,
                  rw_ref, rb_ref, lg_ref, lb_ref,
                  out_ref, *, t, n, f, c):
    X = x_ref[0]
    M = m_ref[:, :]

    u1 = u1_ref[:, :]
    lhs_tf = jnp.concatenate(
        [_dot(u1, X[ti * n:(ti + 1) * n, :]) for ti in range(t)], axis=0)
    lhs = _dot(lhs_tf, u2_ref[:, :])
    rhs_flat = _dot(X, u3_ref[:, :])
    E = jnp.concatenate(
        [_dot(lhs, rhs_flat[si * n:(si + 1) * n, :]) for si in range(t)],
        axis=1)
    Eatt = _dot(ve_ref[:, :], jax.nn.sigmoid(E + be_ref[:, :]))
    Eatt = _softmax0(Eatt)

    Xt = jnp.concatenate([
        sum(Eatt[ti:ti + 1, si:si + 1] * X[ti * n:(ti + 1) * n, :]
            for ti in range(t))
        for si in range(t)], axis=0)

    lhs2 = sum(w1_ref[si:si + 1, 0:1] * Xt[si * n:(si + 1) * n, :]
               for si in range(t))
    lhs2b = _dot(lhs2, w2_ref[:, :])
    rhs2_flat = _dot(Xt, w3_ref[:, :])
    rhs2 = jnp.concatenate(
        [rhs2_flat[ti * n:(ti + 1) * n, :] for ti in range(t)], axis=1)
    P = _dot(lhs2b, rhs2, dims=(((1,), (1,)), ((), ())))
    S = _dot(vs_ref[:, :], jax.nn.sigmoid(P + bs_ref[:, :]))
    S = _softmax0(S)

    eye = (jax.lax.broadcasted_iota(jnp.int32, (n, n), 0) ==
           jax.lax.broadcasted_iota(jnp.int32, (n, n), 1)).astype(_F32)
    diag = jnp.sum(S * eye, axis=1, keepdims=True)
    diag_full = jnp.concatenate([diag] * t, axis=0)
    A1 = M * S

    T0 = diag_full * Xt
    o = _dot(T0, cw0_ref[:, :])
    T1 = jnp.concatenate(
        [_dot(A1, T0[ti * n:(ti + 1) * n, :]) for ti in range(t)], axis=0)
    o = o + _dot(T1, cw1_ref[:, :])
    T2 = jnp.concatenate(
        [_dot(M, T1[ti * n:(ti + 1) * n, :]) for ti in range(t)], axis=0)
    T2 = 2.0 * T2 - T0
    o = o + _dot(T2, cw2_ref[:, :]) + cb_ref[:, :]
    Xh = jax.nn.relu(o)

    Z0 = _dot(Xh, tw0_ref[:, :])
    Z1 = _dot(Xh, tw1_ref[:, :])
    Z2 = _dot(Xh, tw2_ref[:, :])
    zpad = jnp.zeros((n, c), _F32)
    tc = (Z1 + jnp.concatenate([zpad, Z0[:(t - 1) * n, :]], axis=0)
          + jnp.concatenate([Z2[n:, :], zpad], axis=0) + tb_ref[:, :])
    rc = _dot(X, rw_ref[:, :]) + rb_ref[:, :]
    z = jax.nn.relu(rc + tc)
    mu = jnp.mean(z, axis=1, keepdims=True)
    zc = z - mu
    var = jnp.mean(zc * zc, axis=1, keepdims=True)
    z = zc * jax.lax.rsqrt(var + 1e-5) * lg_ref[:, :] + lb_ref[:, :]
    out_ref[0] = z


def _run_block(X, M, p, t, n, c):
    b = X.shape[0]
    f = X.shape[2]
    prm = [
        p['U1'].reshape(1, n), p['U2'], p['U3'].reshape(f, 1),
        p['be'][0], p['Ve'],
        p['W1'].reshape(t, 1), p['W2'], p['W3'].reshape(f, 1),
        p['bs'][0], p['Vs'],
        p['cheb_w'][0], p['cheb_w'][1], p['cheb_w'][2],
        p['cheb_b'].reshape(1, c),
        p['time_w'][:, :, 0, 0].T, p['time_w'][:, :, 0, 1].T,
        p['time_w'][:, :, 0, 2].T, p['time_b'].reshape(1, c),
        p['res_w'][:, :, 0, 0].T, p['res_b'].reshape(1, c),
        p['ln_g'].reshape(1, c), p['ln_b'].reshape(1, c),
    ]
    full = lambda a: pl.BlockSpec(a.shape, lambda i: (0,) * a.ndim)
    in_specs = ([pl.BlockSpec((1, t * n, f), lambda i: (i, 0, 0)),
                 pl.BlockSpec((n, n), lambda i: (0, 0))] +
                [full(a) for a in prm])
    return pl.pallas_call(
        partial(_block_kernel, t=t, n=n, f=f, c=c),
        grid=(b,),
        in_specs=in_specs,
        out_specs=pl.BlockSpec((1, t * n, c), lambda i: (i, 0, 0)),
        out_shape=jax.ShapeDtypeStruct((b, t * n, c), _F32),
        compiler_params=pltpu.CompilerParams(
            dimension_semantics=("parallel",)),
    )(X, M, *prm)


def _final_kernel(x_ref, fw_ref, fcw_ref, fcb_ref, out_ref, *, t, n, c):
    X = x_ref[0]
    fw = fw_ref[:, :]
    o = sum(_dot(X[ti * n:(ti + 1) * n, :], fw[:, ti:ti + 1])
            for ti in range(t))
    y = _dot(fcw_ref[:, :], o) + fcb_ref[:, :]
    out_ref[0] = jax.nn.sigmoid(y)


def _run_final(X, params, t, n, c):
    b = X.shape[0]
    fw = params['final_w'][0, :, 0, :].T
    fcb = (params['fc_b'] +
           params['final_b'][0] * params['fc_w'].sum(axis=1)).reshape(n, 1)
    full = lambda a: pl.BlockSpec(a.shape, lambda i: (0,) * a.ndim)
    out = pl.pallas_call(
        partial(_final_kernel, t=t, n=n, c=c),
        grid=(b,),
        in_specs=[pl.BlockSpec((1, t * n, c), lambda i: (i, 0, 0)),
                  full(fw), full(params['fc_w']), full(fcb)],
        out_specs=pl.BlockSpec((1, n, 1), lambda i: (i, 0, 0)),
        out_shape=jax.ShapeDtypeStruct((b, n, 1), _F32),
        compiler_params=pltpu.CompilerParams(
            dimension_semantics=("parallel",)),
    )(X, fw, params['fc_w'], fcb)
    return jnp.transpose(out, (0, 2, 1))


def kernel(x, edge_index, params):
    b, t, n, f = x.shape
    c = params['blocks'][0]['cheb_b'].shape[0]
    M = _build_m(edge_index, n)
    X = x.reshape(b, t * n, f)
    for p in params['blocks']:
        X = _run_block(X, M, p, t, n, c)
    return _run_final(X, params, t, n, c)


# speedup vs baseline: 34.5430x; 1.1197x over previous
import jax
import jax.numpy as jnp
from functools import partial
from jax.experimental import pallas as pl
from jax.experimental.pallas import tpu as pltpu
from jax.experimental.pallas import tpu_sc as plsc

_F32 = jnp.float32
_HI = jax.lax.Precision.HIGHEST


def _dot(a, b, dims=None, precision=_HI):
    if dims is None:
        dims = (((1,), (0,)), ((), ()))
    return jax.lax.dot_general(a, b, dimension_numbers=dims,
                               precision=precision,
                               preferred_element_type=_F32)


def _softmax0(x):
    m = jnp.max(x, axis=0, keepdims=True)
    e = jnp.exp(x - m)
    return e / jnp.sum(e, axis=0, keepdims=True)


_NC, _NS, _LANES = 2, 16, 16
_ROWW = 128


def _cnt_sc_kernel(rows_hbm, cols_hbm, onehot_hbm, out_hbm,
                   rows_v, cols_v, buf_v, ridx_v, gidx_v, zbuf_v, shared,
                   *, n, e_per_w):
    cid = jax.lax.axis_index("c")
    sid = jax.lax.axis_index("s")
    wid = cid * _NS + sid
    nrows = (n * n) // _ROWW
    sl = nrows // _NS
    zn = zbuf_v.shape[0]
    groups = e_per_w // _LANES

    z16 = jnp.zeros((_LANES,), _F32)
    for r in range(zn):
        for q in range(_ROWW // _LANES):
            zbuf_v[r, pl.ds(q * _LANES, _LANES)] = z16
    for k in range(sl // zn):
        pltpu.sync_copy(zbuf_v, shared.at[pl.ds(sid * sl + k * zn, zn)])

    pltpu.sync_copy(rows_hbm.at[wid], rows_v)
    pltpu.sync_copy(cols_hbm.at[wid], cols_v)

    for g in range(groups):
        rr = rows_v[pl.ds(g * _LANES, _LANES)]
        cc = cols_v[pl.ds(g * _LANES, _LANES)]
        flat = rr * n + cc
        rident = jax.lax.shift_right_logical(flat, 7)
        lane = jax.lax.bitwise_and(flat, _ROWW - 1)
        sel = jnp.where(rr != cc, lane, _ROWW)
        ridx_v[g // 8, pl.ds((g % 8) * _LANES, _LANES)] = rident
        gidx_v[g // 8, pl.ds((g % 8) * _LANES, _LANES)] = sel

    for j in range(e_per_w // 128):
        pltpu.sync_copy(onehot_hbm.at[gidx_v.at[j]],
                        buf_v.at[pl.ds(j * 128, 128)])
    plsc.subcore_barrier()
    for j in range(e_per_w // 128):
        pltpu.sync_copy(buf_v.at[pl.ds(j * 128, 128)],
                        shared.at[ridx_v.at[j]], add=True)
    plsc.subcore_barrier()
    pltpu.sync_copy(shared.at[pl.ds(sid * sl, sl)],
                    out_hbm.at[cid, pl.ds(sid * sl, sl)])


def _build_cnt(edge_index, n):
    e = edge_index.shape[1]
    nw = _NC * _NS
    e_per_w = e // nw
    nrows = (n * n) // _ROWW
    rows32 = edge_index[0].reshape(nw, e_per_w)
    cols32 = edge_index[1].reshape(nw, e_per_w)
    onehot = jnp.concatenate(
        [jnp.eye(_ROWW, dtype=_F32), jnp.zeros((8, _ROWW), _F32)], axis=0)
    mesh = plsc.VectorSubcoreMesh(core_axis_name="c", subcore_axis_name="s")
    k = partial(
        pl.kernel,
        mesh=mesh,
        out_type=jax.ShapeDtypeStruct((_NC, nrows, _ROWW), _F32),
        scratch_types=[
            pltpu.VMEM((e_per_w,), jnp.int32),
            pltpu.VMEM((e_per_w,), jnp.int32),
            pltpu.VMEM((e_per_w, _ROWW), _F32),
            pltpu.VMEM((e_per_w // 128, 128), jnp.int32),
            pltpu.VMEM((e_per_w // 128, 128), jnp.int32),
            pltpu.VMEM((64, _ROWW), _F32),
            pltpu.VMEM_SHARED((nrows, _ROWW), _F32),
        ],
    )(partial(_cnt_sc_kernel, n=n, e_per_w=e_per_w))
    cnt2 = k(rows32, cols32, onehot)
    return cnt2.reshape(_NC, n, n)


def _mfin_kernel(cnt_ref, m_ref, *, n):
    cnt = cnt_ref[0] + cnt_ref[1]
    deg = jnp.sum(cnt, axis=1, keepdims=True)
    dis = jnp.where(deg > 0, jax.lax.rsqrt(jnp.maximum(deg, 1e-12)), 0.0)
    eye = (jax.lax.broadcasted_iota(jnp.int32, (n, n), 0) ==
           jax.lax.broadcasted_iota(jnp.int32, (n, n), 1)).astype(_F32)
    dis_row = _dot(dis, eye, dims=(((0,), (0,)), ((), ())))
    m_ref[:, :] = -(cnt * dis) * dis_row


def _build_m(edge_index, n):
    cnt2 = _build_cnt(edge_index, n)
    return pl.pallas_call(
        partial(_mfin_kernel, n=n),
        out_shape=jax.ShapeDtypeStruct((n, n), _F32),
    )(cnt2)


def _block_kernel(x_ref, m_ref,
                  u1_ref, u2_ref, u3_ref, be_ref, ve_ref,
                  w1_ref, w2_ref, w3_ref, bs_ref, vs_ref,
                  cw0_ref, cw1_ref, cw2_ref, cb_ref,
                  tw0_ref, tw1_ref, tw2_ref, tb_ref,
                  rw_ref, rb_ref, lg_ref, lb_ref,
                  out_ref, *, t, n, f, c):
    X = x_ref[0]
    M = m_ref[:, :]

    u1 = u1_ref[:, :]
    lhs_tf = jnp.concatenate(
        [_dot(u1, X[ti * n:(ti + 1) * n, :]) for ti in range(t)], axis=0)
    lhs = _dot(lhs_tf, u2_ref[:, :])
    rhs_flat = _dot(X, u3_ref[:, :])
    E = jnp.concatenate(
        [_dot(lhs, rhs_flat[si * n:(si + 1) * n, :]) for si in range(t)],
        axis=1)
    Eatt = _dot(ve_ref[:, :], jax.nn.sigmoid(E + be_ref[:, :]))
    Eatt = _softmax0(Eatt)

    Xt = jnp.concatenate([
        sum(Eatt[ti:ti + 1, si:si + 1] * X[ti * n:(ti + 1) * n, :]
            for ti in range(t))
        for si in range(t)], axis=0)

    lhs2 = sum(w1_ref[si:si + 1, 0:1] * Xt[si * n:(si + 1) * n, :]
               for si in range(t))
    lhs2b = _dot(lhs2, w2_ref[:, :])
    rhs2_flat = _dot(Xt, w3_ref[:, :])
    rhs2 = jnp.concatenate(
        [rhs2_flat[ti * n:(ti + 1) * n, :] for ti in range(t)], axis=1)
    P = _dot(lhs2b, rhs2, dims=(((1,), (1,)), ((), ())))
    S = _dot(vs_ref[:, :], jax.nn.sigmoid(P + bs_ref[:, :]))
    S = _softmax0(S)

    eye = (jax.lax.broadcasted_iota(jnp.int32, (n, n), 0) ==
           jax.lax.broadcasted_iota(jnp.int32, (n, n), 1)).astype(_F32)
    diag = jnp.sum(S * eye, axis=1, keepdims=True)
    diag_full = jnp.concatenate([diag] * t, axis=0)
    A1 = M * S

    T0 = diag_full * Xt
    o = _dot(T0, cw0_ref[:, :])
    T0nl = jnp.concatenate(
        [T0[ti * n:(ti + 1) * n, :] for ti in range(t)], axis=1)
    T1nl = _dot(A1, T0nl)
    T1 = jnp.concatenate(
        [T1nl[:, ti * f:(ti + 1) * f] for ti in range(t)], axis=0)
    o = o + _dot(T1, cw1_ref[:, :])
    T2nl = 2.0 * _dot(M, T1nl) - T0nl
    T2 = jnp.concatenate(
        [T2nl[:, ti * f:(ti + 1) * f] for ti in range(t)], axis=0)
    o = o + _dot(T2, cw2_ref[:, :]) + cb_ref[:, :]
    Xh = jax.nn.relu(o)

    Z0 = _dot(Xh, tw0_ref[:, :])
    Z1 = _dot(Xh, tw1_ref[:, :])
    Z2 = _dot(Xh, tw2_ref[:, :])
    zpad = jnp.zeros((n, c), _F32)
    tc = (Z1 + jnp.concatenate([zpad, Z0[:(t - 1) * n, :]], axis=0)
          + jnp.concatenate([Z2[n:, :], zpad], axis=0) + tb_ref[:, :])
    rc = _dot(X, rw_ref[:, :]) + rb_ref[:, :]
    z = jax.nn.relu(rc + tc)
    mu = jnp.mean(z, axis=1, keepdims=True)
    zc = z - mu
    var = jnp.mean(zc * zc, axis=1, keepdims=True)
    z = zc * jax.lax.rsqrt(var + 1e-5) * lg_ref[:, :] + lb_ref[:, :]
    out_ref[0] = z


def _run_block(X, M, p, t, n, c):
    b = X.shape[0]
    f = X.shape[2]
    prm = [
        p['U1'].reshape(1, n), p['U2'], p['U3'].reshape(f, 1),
        p['be'][0], p['Ve'],
        p['W1'].reshape(t, 1), p['W2'], p['W3'].reshape(f, 1),
        p['bs'][0], p['Vs'],
        p['cheb_w'][0], p['cheb_w'][1], p['cheb_w'][2],
        p['cheb_b'].reshape(1, c),
        p['time_w'][:, :, 0, 0].T, p['time_w'][:, :, 0, 1].T,
        p['time_w'][:, :, 0, 2].T, p['time_b'].reshape(1, c),
        p['res_w'][:, :, 0, 0].T, p['res_b'].reshape(1, c),
        p['ln_g'].reshape(1, c), p['ln_b'].reshape(1, c),
    ]
    full = lambda a: pl.BlockSpec(a.shape, lambda i: (0,) * a.ndim)
    in_specs = ([pl.BlockSpec((1, t * n, f), lambda i: (i, 0, 0)),
                 pl.BlockSpec((n, n), lambda i: (0, 0))] +
                [full(a) for a in prm])
    return pl.pallas_call(
        partial(_block_kernel, t=t, n=n, f=f, c=c),
        grid=(b,),
        in_specs=in_specs,
        out_specs=pl.BlockSpec((1, t * n, c), lambda i: (i, 0, 0)),
        out_shape=jax.ShapeDtypeStruct((b, t * n, c), _F32),
        compiler_params=pltpu.CompilerParams(
            dimension_semantics=("parallel",)),
    )(X, M, *prm)


def _final_kernel(x_ref, fw_ref, fcw_ref, fcb_ref, out_ref, *, t, n, c):
    X = x_ref[0]
    fw = fw_ref[:, :]
    o = sum(_dot(X[ti * n:(ti + 1) * n, :], fw[:, ti:ti + 1])
            for ti in range(t))
    y = _dot(fcw_ref[:, :], o) + fcb_ref[:, :]
    out_ref[0] = jax.nn.sigmoid(y)


def _run_final(X, params, t, n, c):
    b = X.shape[0]
    fw = params['final_w'][0, :, 0, :].T
    fcb = (params['fc_b'] +
           params['final_b'][0] * params['fc_w'].sum(axis=1)).reshape(n, 1)
    full = lambda a: pl.BlockSpec(a.shape, lambda i: (0,) * a.ndim)
    out = pl.pallas_call(
        partial(_final_kernel, t=t, n=n, c=c),
        grid=(b,),
        in_specs=[pl.BlockSpec((1, t * n, c), lambda i: (i, 0, 0)),
                  full(fw), full(params['fc_w']), full(fcb)],
        out_specs=pl.BlockSpec((1, n, 1), lambda i: (i, 0, 0)),
        out_shape=jax.ShapeDtypeStruct((b, n, 1), _F32),
        compiler_params=pltpu.CompilerParams(
            dimension_semantics=("parallel",)),
    )(X, fw, params['fc_w'], fcb)
    return jnp.transpose(out, (0, 2, 1))


def kernel(x, edge_index, params):
    b, t, n, f = x.shape
    c = params['blocks'][0]['cheb_b'].shape[0]
    M = _build_m(edge_index, n)
    X = x.reshape(b, t * n, f)
    for p in params['blocks']:
        X = _run_block(X, M, p, t, n, c)
    return _run_final(X, params, t, n, c)


# speedup vs baseline: 37.5756x; 1.0878x over previous
import jax
import jax.numpy as jnp
from functools import partial
from jax.experimental import pallas as pl
from jax.experimental.pallas import tpu as pltpu
from jax.experimental.pallas import tpu_sc as plsc

_F32 = jnp.float32
_HI = jax.lax.Precision.HIGHEST


def _dot(a, b, dims=None, precision=_HI):
    if dims is None:
        dims = (((1,), (0,)), ((), ()))
    return jax.lax.dot_general(a, b, dimension_numbers=dims,
                               precision=precision,
                               preferred_element_type=_F32)


def _softmax0(x):
    m = jnp.max(x, axis=0, keepdims=True)
    e = jnp.exp(x - m)
    return e / jnp.sum(e, axis=0, keepdims=True)


_NC, _NS, _LANES = 2, 16, 16
_ROWW = 128


def _cnt_sc_kernel(rows_hbm, cols_hbm, onehot_hbm, out_hbm,
                   rows_v, cols_v, buf_v, ridx_v, gidx_v, zbuf_v, shared,
                   *, n, e_per_w):
    cid = jax.lax.axis_index("c")
    sid = jax.lax.axis_index("s")
    wid = cid * _NS + sid
    nrows = (n * n) // _ROWW
    sl = nrows // _NS
    zn = zbuf_v.shape[0]
    groups = e_per_w // _LANES

    z16 = jnp.zeros((_LANES,), _F32)
    for r in range(zn):
        for q in range(_ROWW // _LANES):
            zbuf_v[r, pl.ds(q * _LANES, _LANES)] = z16
    for k in range(sl // zn):
        pltpu.sync_copy(zbuf_v, shared.at[pl.ds(sid * sl + k * zn, zn)])

    pltpu.sync_copy(rows_hbm.at[wid], rows_v)
    pltpu.sync_copy(cols_hbm.at[wid], cols_v)

    for g in range(groups):
        rr = rows_v[pl.ds(g * _LANES, _LANES)]
        cc = cols_v[pl.ds(g * _LANES, _LANES)]
        flat = rr * n + cc
        rident = jax.lax.shift_right_logical(flat, 7)
        lane = jax.lax.bitwise_and(flat, _ROWW - 1)
        sel = jnp.where(rr != cc, lane, _ROWW)
        ridx_v[g // 8, pl.ds((g % 8) * _LANES, _LANES)] = rident
        gidx_v[g // 8, pl.ds((g % 8) * _LANES, _LANES)] = sel

    for j in range(e_per_w // 128):
        pltpu.sync_copy(onehot_hbm.at[gidx_v.at[j]],
                        buf_v.at[pl.ds(j * 128, 128)])
    plsc.subcore_barrier()
    for j in range(e_per_w // 128):
        pltpu.sync_copy(buf_v.at[pl.ds(j * 128, 128)],
                        shared.at[ridx_v.at[j]], add=True)
    plsc.subcore_barrier()
    pltpu.sync_copy(shared.at[pl.ds(sid * sl, sl)],
                    out_hbm.at[cid, pl.ds(sid * sl, sl)])


def _build_cnt(edge_index, n):
    e = edge_index.shape[1]
    nw = _NC * _NS
    e_per_w = e // nw
    nrows = (n * n) // _ROWW
    rows32 = edge_index[0].reshape(nw, e_per_w)
    cols32 = edge_index[1].reshape(nw, e_per_w)
    onehot = jnp.concatenate(
        [jnp.eye(_ROWW, dtype=_F32), jnp.zeros((8, _ROWW), _F32)], axis=0)
    mesh = plsc.VectorSubcoreMesh(core_axis_name="c", subcore_axis_name="s")
    k = partial(
        pl.kernel,
        mesh=mesh,
        out_type=jax.ShapeDtypeStruct((_NC, nrows, _ROWW), _F32),
        scratch_types=[
            pltpu.VMEM((e_per_w,), jnp.int32),
            pltpu.VMEM((e_per_w,), jnp.int32),
            pltpu.VMEM((e_per_w, _ROWW), _F32),
            pltpu.VMEM((e_per_w // 128, 128), jnp.int32),
            pltpu.VMEM((e_per_w // 128, 128), jnp.int32),
            pltpu.VMEM((64, _ROWW), _F32),
            pltpu.VMEM_SHARED((nrows, _ROWW), _F32),
        ],
    )(partial(_cnt_sc_kernel, n=n, e_per_w=e_per_w))
    cnt2 = k(rows32, cols32, onehot)
    return cnt2.reshape(_NC, n, n)


def _mfin_kernel(cnt_ref, m_ref, *, n):
    cnt = cnt_ref[0] + cnt_ref[1]
    deg = jnp.sum(cnt, axis=1, keepdims=True)
    dis = jnp.where(deg > 0, jax.lax.rsqrt(jnp.maximum(deg, 1e-12)), 0.0)
    eye = (jax.lax.broadcasted_iota(jnp.int32, (n, n), 0) ==
           jax.lax.broadcasted_iota(jnp.int32, (n, n), 1)).astype(_F32)
    dis_row = _dot(dis, eye, dims=(((0,), (0,)), ((), ())))
    m_ref[:, :] = -(cnt * dis) * dis_row


def _build_m(edge_index, n):
    cnt2 = _build_cnt(edge_index, n)
    return pl.pallas_call(
        partial(_mfin_kernel, n=n),
        out_shape=jax.ShapeDtypeStruct((n, n), _F32),
    )(cnt2)


def _block_kernel(x_ref, m_ref,
                  u1_ref, u2_ref, u3_ref, be_ref, ve_ref,
                  w1_ref, w2_ref, w3_ref, bs_ref, vs_ref,
                  cw0_ref, cb_ref,
                  tw0_ref, tb_ref,
                  rw_ref, rb_ref, lg_ref, lb_ref,
                  out_ref, *, t, n, f, c):
    X = x_ref[0]
    M = m_ref[:, :]

    u1 = u1_ref[:, :]
    lhs_tf = jnp.concatenate(
        [_dot(u1, X[ti * n:(ti + 1) * n, :]) for ti in range(t)], axis=0)
    lhs = _dot(lhs_tf, u2_ref[:, :])
    rhs_flat = _dot(X, u3_ref[:, :])
    E = jnp.concatenate(
        [_dot(lhs, rhs_flat[si * n:(si + 1) * n, :]) for si in range(t)],
        axis=1)
    Eatt = _dot(ve_ref[:, :], jax.nn.sigmoid(E + be_ref[:, :]))
    Eatt = _softmax0(Eatt)

    Xt = jnp.concatenate([
        sum(Eatt[ti:ti + 1, si:si + 1] * X[ti * n:(ti + 1) * n, :]
            for ti in range(t))
        for si in range(t)], axis=0)

    lhs2 = sum(w1_ref[si:si + 1, 0:1] * Xt[si * n:(si + 1) * n, :]
               for si in range(t))
    lhs2b = _dot(lhs2, w2_ref[:, :])
    rhs2_flat = _dot(Xt, w3_ref[:, :])
    rhs2 = jnp.concatenate(
        [rhs2_flat[ti * n:(ti + 1) * n, :] for ti in range(t)], axis=1)
    P = _dot(lhs2b, rhs2, dims=(((1,), (1,)), ((), ())))
    S = _dot(vs_ref[:, :], jax.nn.sigmoid(P + bs_ref[:, :]))
    S = _softmax0(S)

    eye = (jax.lax.broadcasted_iota(jnp.int32, (n, n), 0) ==
           jax.lax.broadcasted_iota(jnp.int32, (n, n), 1)).astype(_F32)
    diag = jnp.sum(S * eye, axis=1, keepdims=True)
    diag_full = jnp.concatenate([diag] * t, axis=0)
    A1 = M * S

    T0 = diag_full * Xt
    T0nl = jnp.concatenate(
        [T0[ti * n:(ti + 1) * n, :] for ti in range(t)], axis=1)
    T1nl = _dot(A1, T0nl)
    T1 = jnp.concatenate(
        [T1nl[:, ti * f:(ti + 1) * f] for ti in range(t)], axis=0)
    T2nl = 2.0 * _dot(M, T1nl) - T0nl
    T2 = jnp.concatenate(
        [T2nl[:, ti * f:(ti + 1) * f] for ti in range(t)], axis=0)
    T012 = jnp.concatenate([T0, T1, T2], axis=1)
    o = _dot(T012, cw0_ref[:, :]) + cb_ref[:, :]
    Xh = jax.nn.relu(o)

    Z = _dot(Xh, tw0_ref[:, :])
    Z0 = Z[:, :c]
    Z1 = Z[:, c:2 * c]
    Z2 = Z[:, 2 * c:]
    zpad = jnp.zeros((n, c), _F32)
    tc = (Z1 + jnp.concatenate([zpad, Z0[:(t - 1) * n, :]], axis=0)
          + jnp.concatenate([Z2[n:, :], zpad], axis=0) + tb_ref[:, :])
    rc = _dot(X, rw_ref[:, :]) + rb_ref[:, :]
    z = jax.nn.relu(rc + tc)
    mu = jnp.mean(z, axis=1, keepdims=True)
    zc = z - mu
    var = jnp.mean(zc * zc, axis=1, keepdims=True)
    z = zc * jax.lax.rsqrt(var + 1e-5) * lg_ref[:, :] + lb_ref[:, :]
    out_ref[0] = z


def _run_block(X, M, p, t, n, c):
    b = X.shape[0]
    f = X.shape[2]
    prm = [
        p['U1'].reshape(1, n), p['U2'], p['U3'].reshape(f, 1),
        p['be'][0], p['Ve'],
        p['W1'].reshape(t, 1), p['W2'], p['W3'].reshape(f, 1),
        p['bs'][0], p['Vs'],
        jnp.concatenate([p['cheb_w'][0], p['cheb_w'][1], p['cheb_w'][2]],
                        axis=0),
        p['cheb_b'].reshape(1, c),
        jnp.concatenate([p['time_w'][:, :, 0, 0].T, p['time_w'][:, :, 0, 1].T,
                         p['time_w'][:, :, 0, 2].T], axis=1),
        p['time_b'].reshape(1, c),
        p['res_w'][:, :, 0, 0].T, p['res_b'].reshape(1, c),
        p['ln_g'].reshape(1, c), p['ln_b'].reshape(1, c),
    ]
    full = lambda a: pl.BlockSpec(a.shape, lambda i: (0,) * a.ndim)
    in_specs = ([pl.BlockSpec((1, t * n, f), lambda i: (i, 0, 0)),
                 pl.BlockSpec((n, n), lambda i: (0, 0))] +
                [full(a) for a in prm])
    return pl.pallas_call(
        partial(_block_kernel, t=t, n=n, f=f, c=c),
        grid=(b,),
        in_specs=in_specs,
        out_specs=pl.BlockSpec((1, t * n, c), lambda i: (i, 0, 0)),
        out_shape=jax.ShapeDtypeStruct((b, t * n, c), _F32),
        compiler_params=pltpu.CompilerParams(
            dimension_semantics=("parallel",)),
    )(X, M, *prm)


def _final_kernel(x_ref, fw_ref, fcw_ref, fcb_ref, out_ref, *, t, n, c):
    X = x_ref[0]
    fw = fw_ref[:, :]
    o = sum(_dot(X[ti * n:(ti + 1) * n, :], fw[:, ti:ti + 1])
            for ti in range(t))
    y = _dot(fcw_ref[:, :], o) + fcb_ref[:, :]
    out_ref[0] = jax.nn.sigmoid(y)


def _run_final(X, params, t, n, c):
    b = X.shape[0]
    fw = params['final_w'][0, :, 0, :].T
    fcb = (params['fc_b'] +
           params['final_b'][0] * params['fc_w'].sum(axis=1)).reshape(n, 1)
    full = lambda a: pl.BlockSpec(a.shape, lambda i: (0,) * a.ndim)
    out = pl.pallas_call(
        partial(_final_kernel, t=t, n=n, c=c),
        grid=(b,),
        in_specs=[pl.BlockSpec((1, t * n, c), lambda i: (i, 0, 0)),
                  full(fw), full(params['fc_w']), full(fcb)],
        out_specs=pl.BlockSpec((1, n, 1), lambda i: (i, 0, 0)),
        out_shape=jax.ShapeDtypeStruct((b, n, 1), _F32),
        compiler_params=pltpu.CompilerParams(
            dimension_semantics=("parallel",)),
    )(X, fw, params['fc_w'], fcb)
    return jnp.transpose(out, (0, 2, 1))


def kernel(x, edge_index, params):
    b, t, n, f = x.shape
    c = params['blocks'][0]['cheb_b'].shape[0]
    M = _build_m(edge_index, n)
    X = x.reshape(b, t * n, f)
    for p in params['blocks']:
        X = _run_block(X, M, p, t, n, c)
    return _run_final(X, params, t, n, c)
